# baseline (speedup 1.0000x reference)
import functools

import jax
import jax.numpy as jnp
import numpy as np
from jax import lax
from jax.experimental import pallas as pl
from jax.experimental.pallas import tpu as pltpu
from jax.experimental.pallas import tpu_sc as plsc

N = 10000
E = 160000
D = 128
T = 3
R = 5
H = 8
DK = 16
MAXT = 240

NC = 2
NS = 16
NW = NC * NS
CE = 128
NCHUNK = E // CE
BASE_CH = NCHUNK // NW
EXTRA_W = NCHUNK - BASE_CH * NW
CE2 = 64
NCHUNK2 = E // CE2
BASE_CH2 = NCHUNK2 // NW
EXTRA_W2 = NCHUNK2 - BASE_CH2 * NW
RPS = N // NS

BN = 1000
GRID = N // BN

@functools.cache
def _mesh():
    return plsc.VectorSubcoreMesh(core_axis_name="c", subcore_axis_name="s",
                                  num_cores=NC, num_subcores=NS)


def _rte_table():
    pos = np.arange(MAXT, dtype=np.float32)[:, None]
    div = np.exp(np.arange(0, D, 2, dtype=np.float32) * -(np.log(10000.0) / D))
    tab = np.zeros((MAXT, D), dtype=np.float32)
    tab[:, 0::2] = np.sin(pos * div)
    tab[:, 1::2] = np.cos(pos * div)
    return tab / np.sqrt(D)


_RTE = _rte_table()


def _typed_lin(xb, m_ref, w_ref, b_all):
    acc = jnp.zeros_like(xb)
    for t in range(T):
        y = jnp.dot(xb, w_ref[t], preferred_element_type=jnp.float32)
        acc = acc + m_ref[:, t:t + 1] * (y + b_all[t][None, :])
    return acc


def _adapt_body(x_ref, m_ref, w_ref, b_ref, o_ref):
    o_ref[...] = jnp.tanh(_typed_lin(x_ref[...], m_ref, w_ref, b_ref[...]))


def _prep_body(x_ref, m_ref, wk_ref, bk_ref, wq_ref, bq_ref, wv_ref, bv_ref,
               a_ref, mm_ref, kn_ref, qr_ref, vmr_ref):
    xb = x_ref[...]
    kn_ref[...] = _typed_lin(xb, m_ref, wk_ref, bk_ref[...])
    qb = _typed_lin(xb, m_ref, wq_ref, bq_ref[...])
    vb = _typed_lin(xb, m_ref, wv_ref, bv_ref[...])
    for r in range(R):
        qr_ref[r] = jnp.dot(qb, a_ref[r], preferred_element_type=jnp.float32)
        vmr_ref[r] = jnp.dot(vb, mm_ref[r], preferred_element_type=jnp.float32)


def _inv_body(d0_ref, d1_ref, o_ref):
    o_ref[...] = 1.0 / (d0_ref[...] + d1_ref[...] + 1e-16)


def _post_body(a0_ref, a1_ref, x_ref, m_ref, wa_ref, ba_ref, al_ref, o_ref):
    g = a0_ref[...] + a1_ref[...]
    g = 0.5 * g * (1.0 + lax.erf(g * np.float32(1.0 / np.sqrt(2.0))))
    ba = ba_ref[...]
    al = al_ref[...]
    acc = jnp.zeros_like(g)
    alph = jnp.zeros_like(g)
    for t in range(T):
        y = jnp.dot(g, wa_ref[t], preferred_element_type=jnp.float32)
        mt = m_ref[:, t:t + 1]
        acc = acc + mt * (y + ba[t][None, :])
        alph = alph + mt * al[t][None, :]
    o_ref[...] = acc * alph + x_ref[...] * (1.0 - alph)


def _row_spec(shape):
    if shape[0] == N:
        return pl.BlockSpec((BN,) + shape[1:],
                            lambda i: (i,) + (0,) * (len(shape) - 1))
    return pl.BlockSpec((shape[0], BN) + shape[2:], lambda i: (0, i, 0))


def _full_spec(shape):
    return pl.BlockSpec(shape, lambda i: (0,) * len(shape))


def _tc_call(body, in_arrays, out_shapes, row_in, row_out):
    in_specs = [(_row_spec(a.shape) if rb else _full_spec(a.shape))
                for a, rb in zip(in_arrays, row_in)]
    out_specs = [(_row_spec(s.shape) if rb else _full_spec(s.shape))
                 for s, rb in zip(out_shapes, row_out)]
    return pl.pallas_call(
        body,
        grid=(GRID,),
        in_specs=in_specs,
        out_specs=out_specs if len(out_specs) > 1 else out_specs[0],
        out_shape=out_shapes if len(out_shapes) > 1 else out_shapes[0],
    )(*in_arrays)


def _worker_id():
    return lax.axis_index("s") * NC + lax.axis_index("c")


def _nchunks(wid):
    return BASE_CH + jnp.where(wid < EXTRA_W, 1, 0)


_IOTA16 = lambda: lax.iota(jnp.int32, 16)


def _sc_index_body(src_hbm, dst_hbm, rt_hbm, tt_hbm, nt16_hbm,
                   ikt_hbm, iq_hbm, ivm_hbm, ivt_hbm,
                   srcb, dstb, rtb, ttb, ntb, iktb, iqb, ivmb, ivtb, sem):
    wid = _worker_id()

    def chunk(i, carry):
        base = (wid + NW * i) * CE
        pltpu.sync_copy(src_hbm.at[pl.ds(base, CE)], srcb)
        pltpu.sync_copy(dst_hbm.at[pl.ds(base, CE)], dstb)
        pltpu.sync_copy(rt_hbm.at[pl.ds(base, CE)], rtb)
        pltpu.sync_copy(tt_hbm.at[pl.ds(base, CE)], ttb)
        pltpu.async_copy(nt16_hbm.at[srcb], ntb, sem).wait()
        for g in range(CE // 16):
            sl = pl.ds(g * 16, 16)
            rows = jnp.full((16,), g * 16, jnp.int32) + _IOTA16()
            ntv = plsc.load_gather(ntb, [rows, jnp.zeros((16,), jnp.int32)])
            ikt = ntv * MAXT + ttb[sl]
            iktb[sl] = ikt
            iqb[sl] = rtb[sl] * N + dstb[sl]
            ivmb[sl] = rtb[sl] * N + srcb[sl]
            ivtb[sl] = rtb[sl] * (T * MAXT) + ikt
        pltpu.sync_copy(iktb, ikt_hbm.at[pl.ds(base, CE)])
        pltpu.sync_copy(iqb, iq_hbm.at[pl.ds(base, CE)])
        pltpu.sync_copy(ivmb, ivm_hbm.at[pl.ds(base, CE)])
        pltpu.sync_copy(ivtb, ivt_hbm.at[pl.ds(base, CE)])
        return carry

    lax.fori_loop(0, _nchunks(wid), chunk, 0)


NQUAD1 = (BASE_CH + 1 + 3) // 4
NQUAD2 = (BASE_CH2 + 1 + 3) // 4


def _sc_att_body(kn_hbm, kt_hbm, qr_hbm, src_hbm, ikt_hbm, iq_hbm, dst_hbm,
                 zero_hbm,
                 ex_hbm, den0_hbm, den1_hbm,
                 srcb0, iktb0, iqb0, dstb0,
                 srcb1, iktb1, iqb1, dstb1,
                 srcb2, iktb2, iqb2, dstb2,
                 srcb3, iktb3, iqb3, dstb3,
                 knb0, ktb0, qrb0, knb1, ktb1, qrb1, exb0, exb1,
                 den_sh,
                 gsem0, gsem1, stsem0, stsem1):
    c = lax.axis_index("c")
    s = lax.axis_index("s")
    wid = s * NC + c
    nch = _nchunks(wid)
    idxs = [(srcb0, iktb0, iqb0, dstb0), (srcb1, iktb1, iqb1, dstb1),
            (srcb2, iktb2, iqb2, dstb2), (srcb3, iktb3, iqb3, dstb3)]
    gbufs = [(knb0, ktb0, qrb0), (knb1, ktb1, qrb1)]
    exs = [exb0, exb1]
    gsems = [gsem0, gsem1]
    stsems = [stsem0, stsem1]
    ihbm = (src_hbm, ikt_hbm, iq_hbm, dst_hbm)

    def load_idx(i, q):
        b = (wid + NW * i) * CE
        for href, vref in zip(ihbm, idxs[q]):
            pltpu.sync_copy(href.at[pl.ds(b, CE)], vref)

    def gathers(q, j):
        srcb, iktb, iqb, _ = idxs[q]
        knb, ktb, qrb = gbufs[j]
        return [pltpu.async_copy(kn_hbm.at[srcb], knb, gsems[j]),
                pltpu.async_copy(kt_hbm.at[iktb], ktb, gsems[j]),
                pltpu.async_copy(qr_hbm.at[iqb], qrb, gsems[j])]

    def compute(j):
        knb, ktb, qrb = gbufs[j]
        exb = exs[j]

        def edge(e2, carry):
            iota = _IOTA16()
            for u in range(2):
                ev = jnp.full((16,), e2 * 2 + u, jnp.int32)
                ex = jnp.zeros((16,), jnp.float32)
                for h in range(H):
                    col = iota + h * DK
                    kv = plsc.load_gather(knb, [ev, col])
                    tv = plsc.load_gather(ktb, [ev, col])
                    qv = plsc.load_gather(qrb, [ev, col])
                    sdot = jnp.sum((kv + tv) * qv)
                    ex = jnp.where(iota == h, sdot, ex)
                plsc.store_scatter(exb, [ev, iota], jnp.exp(ex))
            return carry

        lax.fori_loop(0, CE // 2, edge, 0)

    def store(b, q, j):
        d = pltpu.async_copy(exs[j], ex_hbm.at[pl.ds(b, CE)], stsems[j])
        pltpu.sync_copy(exs[j], den_sh.at[idxs[q][3]], add=True)
        return d

    rsl = pl.ds(s * RPS, RPS)
    pltpu.sync_copy(zero_hbm.at[rsl], den_sh.at[rsl])
    plsc.subcore_barrier()

    def quad(t, carry):
        cc = [4 * t + k for k in range(4)]
        bases = [(wid + NW * c_) * CE for c_ in cc]
        dg = [None] * 4
        ds = [None] * 4

        def when(k):
            return pl.when(cc[k] < nch)

        @when(0)
        def _():
            load_idx(cc[0], 0)
            dg[0] = gathers(0, 0)

        @when(1)
        def _():
            load_idx(cc[1], 1)
            dg[1] = gathers(1, 1)

        @when(0)
        def _():
            for d in dg[0]:
                d.wait()
            compute(0)
            ds[0] = store(bases[0], 0, 0)

        @when(2)
        def _():
            load_idx(cc[2], 2)
            dg[2] = gathers(2, 0)

        @when(1)
        def _():
            for d in dg[1]:
                d.wait()
            compute(1)
            ds[1] = store(bases[1], 1, 1)

        @when(3)
        def _():
            load_idx(cc[3], 3)
            dg[3] = gathers(3, 1)

        @when(0)
        def _():
            ds[0].wait()

        @when(2)
        def _():
            for d in dg[2]:
                d.wait()
            compute(0)
            ds[2] = store(bases[2], 2, 0)

        @when(1)
        def _():
            ds[1].wait()

        @when(3)
        def _():
            for d in dg[3]:
                d.wait()
            compute(1)
            ds[3] = store(bases[3], 3, 1)

        @when(2)
        def _():
            ds[2].wait()

        @when(3)
        def _():
            ds[3].wait()

        return carry

    lax.fori_loop(0, NQUAD1, quad, 0)
    plsc.subcore_barrier()

    @pl.when(c == 0)
    def _():
        pltpu.sync_copy(den_sh.at[rsl], den0_hbm.at[rsl])

    @pl.when(c == 1)
    def _():
        pltpu.sync_copy(den_sh.at[rsl], den1_hbm.at[rsl])


def _sc_agg_body(vmr_hbm, vmt_hbm, ex_hbm, inv_hbm,
                 ivm_hbm, ivt_hbm, dst_hbm, zero_hbm,
                 agg0_hbm, agg1_hbm,
                 ivmb0, ivtb0, dstb0, exb0,
                 ivmb1, ivtb1, dstb1, exb1,
                 ivmb2, ivtb2, dstb2, exb2,
                 ivmb3, ivtb3, dstb3, exb3,
                 vmrb0, vmtb0, invb0, vmrb1, vmtb1, invb1,
                 agg_sh,
                 gsem0, gsem1):
    c = lax.axis_index("c")
    s = lax.axis_index("s")
    wid = s * NC + c
    nch = BASE_CH2 + jnp.where(wid < EXTRA_W2, 1, 0)
    idxs = [(ivmb0, ivtb0, dstb0, exb0), (ivmb1, ivtb1, dstb1, exb1),
            (ivmb2, ivtb2, dstb2, exb2), (ivmb3, ivtb3, dstb3, exb3)]
    gbufs = [(vmrb0, vmtb0, invb0), (vmrb1, vmtb1, invb1)]
    gsems = [gsem0, gsem1]
    ihbm = (ivm_hbm, ivt_hbm, dst_hbm)

    def load_idx(i, q):
        b = (wid + NW * i) * CE2
        for href, vref in zip(ihbm, idxs[q][:3]):
            pltpu.sync_copy(href.at[pl.ds(b, CE2)], vref)
        pltpu.sync_copy(ex_hbm.at[pl.ds(b, CE2)], idxs[q][3])

    def gathers(q, j):
        ivmb, ivtb, dstb, _ = idxs[q]
        vmrb, vmtb, invb = gbufs[j]
        return [pltpu.async_copy(vmr_hbm.at[ivmb], vmrb, gsems[j]),
                pltpu.async_copy(vmt_hbm.at[ivtb], vmtb, gsems[j]),
                pltpu.async_copy(inv_hbm.at[dstb], invb, gsems[j])]

    def compute(q, j):
        vmrb, vmtb, invb = gbufs[j]
        exb = idxs[q][3]

        def edge(e2, carry):
            iota = _IOTA16()
            for u in range(2):
                ev = jnp.full((16,), e2 * 2 + u, jnp.int32)
                exv = plsc.load_gather(exb, [ev, iota])
                attn = exv * plsc.load_gather(invb, [ev, iota])
                for h in range(H):
                    col = iota + h * DK
                    mv = (plsc.load_gather(vmrb, [ev, col])
                          + plsc.load_gather(vmtb, [ev, col]))
                    plsc.store_scatter(vmrb, [ev, col], mv * attn[h])
            return carry

        lax.fori_loop(0, CE2 // 2, edge, 0)

    def store(q, j):
        pltpu.sync_copy(gbufs[j][0], agg_sh.at[idxs[q][2]], add=True)

    rsl = pl.ds(s * RPS, RPS)
    pltpu.sync_copy(zero_hbm.at[rsl], agg_sh.at[rsl])
    plsc.subcore_barrier()

    def quad(t, carry):
        cc = [4 * t + k for k in range(4)]
        dg = [None] * 4

        def when(k):
            return pl.when(cc[k] < nch)

        @when(0)
        def _():
            load_idx(cc[0], 0)
            dg[0] = gathers(0, 0)

        @when(1)
        def _():
            load_idx(cc[1], 1)
            dg[1] = gathers(1, 1)

        @when(0)
        def _():
            for d in dg[0]:
                d.wait()
            compute(0, 0)
            store(0, 0)

        @when(2)
        def _():
            load_idx(cc[2], 2)
            dg[2] = gathers(2, 0)

        @when(1)
        def _():
            for d in dg[1]:
                d.wait()
            compute(1, 1)
            store(1, 1)

        @when(3)
        def _():
            load_idx(cc[3], 3)
            dg[3] = gathers(3, 1)

        @when(2)
        def _():
            for d in dg[2]:
                d.wait()
            compute(2, 0)
            store(2, 0)

        @when(3)
        def _():
            for d in dg[3]:
                d.wait()
            compute(3, 1)
            store(3, 1)

        return carry

    lax.fori_loop(0, NQUAD2, quad, 0)
    plsc.subcore_barrier()

    @pl.when(c == 0)
    def _():
        pltpu.sync_copy(agg_sh.at[rsl], agg0_hbm.at[rsl])

    @pl.when(c == 1)
    def _():
        pltpu.sync_copy(agg_sh.at[rsl], agg1_hbm.at[rsl])


@functools.cache
def _sc_index():
    return pl.kernel(
        _sc_index_body,
        out_type=[jax.ShapeDtypeStruct((E,), jnp.int32)] * 4,
        mesh=_mesh(),
        compiler_params=pltpu.CompilerParams(needs_layout_passes=False, use_tc_tiling_on_sc=False),
        scratch_types=(
            [pltpu.VMEM((CE,), jnp.int32)] * 4
            + [pltpu.VMEM((CE, 16), jnp.int32)]
            + [pltpu.VMEM((CE,), jnp.int32)] * 4
            + [pltpu.SemaphoreType.DMA]
        ),
    )


@functools.cache
def _sc_att():
    return pl.kernel(
        _sc_att_body,
        out_type=[jax.ShapeDtypeStruct((E, 16), jnp.float32),
                  jax.ShapeDtypeStruct((N, 16), jnp.float32),
                  jax.ShapeDtypeStruct((N, 16), jnp.float32)],
        mesh=_mesh(),
        compiler_params=pltpu.CompilerParams(needs_layout_passes=False, use_tc_tiling_on_sc=False),
        scratch_types=(
            [pltpu.VMEM((CE,), jnp.int32)] * 16
            + [pltpu.VMEM((CE, D), jnp.float32)] * 6
            + [pltpu.VMEM((CE, 16), jnp.float32)] * 2
            + [pltpu.VMEM_SHARED((N, 16), jnp.float32)]
            + [pltpu.SemaphoreType.DMA] * 4
        ),
    )


@functools.cache
def _sc_agg():
    return pl.kernel(
        _sc_agg_body,
        out_type=[jax.ShapeDtypeStruct((N, D), jnp.float32),
                  jax.ShapeDtypeStruct((N, D), jnp.float32)],
        mesh=_mesh(),
        compiler_params=pltpu.CompilerParams(needs_layout_passes=False, use_tc_tiling_on_sc=False),
        scratch_types=(
            ([pltpu.VMEM((CE2,), jnp.int32)] * 3
             + [pltpu.VMEM((CE2, 16), jnp.float32)]) * 4
            + ([pltpu.VMEM((CE2, D), jnp.float32)] * 2
               + [pltpu.VMEM((CE2, 16), jnp.float32)]) * 2
            + [pltpu.VMEM_SHARED((N, D), jnp.float32)]
            + [pltpu.SemaphoreType.DMA] * 2
        ),
    )


def _blockdiag(blocks):
    out = jnp.zeros((R, D, D), jnp.float32)
    for h in range(H):
        out = out.at[:, h * DK:(h + 1) * DK, h * DK:(h + 1) * DK].set(blocks[:, h])
    return out


def kernel(node_feature, params, node_type, edge_time, edge_index, edge_type):
    node_type = node_type.astype(jnp.int32)
    src = edge_index[0].astype(jnp.int32)
    dst = edge_index[1].astype(jnp.int32)
    rt = edge_type.astype(jnp.int32)
    tt = edge_time.astype(jnp.int32)

    mask = jnp.pad(jax.nn.one_hot(node_type, T, dtype=jnp.float32),
                   ((0, 0), (0, D - T)))
    nt16 = jnp.tile(node_type[:, None], (1, 16))
    zeros_d = jnp.zeros((N, 16), jnp.float32)
    zeros_a = jnp.zeros((N, D), jnp.float32)

    ikt, iq, ivm, ivt = _sc_index()(src, dst, rt, tt, nt16)

    x = _tc_call(
        _adapt_body,
        [node_feature, mask,
         params['adapt_W'].transpose(0, 2, 1), params['adapt_b']],
        [jax.ShapeDtypeStruct((N, D), jnp.float32)],
        [True, True, False, False], [True])

    for lp in params['layers']:
        rte_proj = _RTE @ lp['rte_W'].T + lp['rte_b']
        ktime = jnp.einsum('md,tod->tmo', rte_proj, lp['Wk']).reshape(T * MAXT, D)
        vtime = jnp.einsum('md,tod->tmo', rte_proj, lp['Wv']).reshape(T * MAXT, D)
        scale = (lp['rel_pri'] / np.sqrt(DK))[:, :, None, None]
        abig = _blockdiag(lp['rel_att'].transpose(0, 1, 3, 2) * scale)
        mbig = _blockdiag(lp['rel_msg'])
        vmt = jnp.einsum('md,rdo->rmo', vtime, mbig).reshape(R * T * MAXT, D)
        alpha = jnp.tile(jax.nn.sigmoid(lp['skip'])[:, None], (1, D))

        kn, qr, vmr = _tc_call(
            _prep_body,
            [x, mask,
             lp['Wk'].transpose(0, 2, 1), lp['bk'],
             lp['Wq'].transpose(0, 2, 1), lp['bq'],
             lp['Wv'].transpose(0, 2, 1), lp['bv'],
             abig, mbig],
            [jax.ShapeDtypeStruct((N, D), jnp.float32),
             jax.ShapeDtypeStruct((R, N, D), jnp.float32),
             jax.ShapeDtypeStruct((R, N, D), jnp.float32)],
            [True, True] + [False] * 8, [True, True, True])

        ex, den0, den1 = _sc_att()(kn, ktime, qr.reshape(R * N, D),
                                   src, ikt, iq, dst, zeros_d)
        inv = _tc_call(_inv_body, [den0, den1],
                       [jax.ShapeDtypeStruct((N, 16), jnp.float32)],
                       [True, True], [True])
        agg0, agg1 = _sc_agg()(vmr.reshape(R * N, D), vmt, ex, inv,
                               ivm, ivt, dst, zeros_a)

        x = _tc_call(
            _post_body,
            [agg0, agg1, x, mask,
             lp['Wa'].transpose(0, 2, 1), lp['ba'], alpha],
            [jax.ShapeDtypeStruct((N, D), jnp.float32)],
            [True, True, True, True, False, False, False], [True])

    return x


# speedup vs baseline: 1.0218x; 1.0218x over previous
import functools

import jax
import jax.numpy as jnp
import numpy as np
from jax import lax
from jax.experimental import pallas as pl
from jax.experimental.pallas import tpu as pltpu
from jax.experimental.pallas import tpu_sc as plsc

N = 10000
E = 160000
D = 128
T = 3
R = 5
H = 8
DK = 16
MAXT = 240

NC = 2
NS = 16
NW = NC * NS
CE = 128
NCHUNK = E // CE
BASE_CH = NCHUNK // NW
EXTRA_W = NCHUNK - BASE_CH * NW
CE2 = 64
NCHUNK2 = E // CE2
BASE_CH2 = NCHUNK2 // NW
EXTRA_W2 = NCHUNK2 - BASE_CH2 * NW
RPS = N // NS

BN = 1000
GRID = N // BN

@functools.cache
def _mesh():
    return plsc.VectorSubcoreMesh(core_axis_name="c", subcore_axis_name="s",
                                  num_cores=NC, num_subcores=NS)


def _rte_table():
    pos = np.arange(MAXT, dtype=np.float32)[:, None]
    div = np.exp(np.arange(0, D, 2, dtype=np.float32) * -(np.log(10000.0) / D))
    tab = np.zeros((MAXT, D), dtype=np.float32)
    tab[:, 0::2] = np.sin(pos * div)
    tab[:, 1::2] = np.cos(pos * div)
    return tab / np.sqrt(D)


_RTE = _rte_table()


def _typed_lin(xb, m_ref, w_ref, b_all):
    acc = jnp.zeros_like(xb)
    for t in range(T):
        y = jnp.dot(xb, w_ref[t], preferred_element_type=jnp.float32)
        acc = acc + m_ref[:, t:t + 1] * (y + b_all[t][None, :])
    return acc


def _adapt_body(x_ref, m_ref, w_ref, b_ref, o_ref):
    o_ref[...] = jnp.tanh(_typed_lin(x_ref[...], m_ref, w_ref, b_ref[...]))


def _prep_body(x_ref, m_ref, wk_ref, bk_ref, wq_ref, bq_ref, wv_ref, bv_ref,
               a_ref, mm_ref, kn_ref, qr_ref, vmr_ref):
    xb = x_ref[...]
    kn_ref[...] = _typed_lin(xb, m_ref, wk_ref, bk_ref[...])
    qb = _typed_lin(xb, m_ref, wq_ref, bq_ref[...])
    vb = _typed_lin(xb, m_ref, wv_ref, bv_ref[...])
    for r in range(R):
        qr_ref[r] = jnp.dot(qb, a_ref[r], preferred_element_type=jnp.float32)
        vmr_ref[r] = jnp.dot(vb, mm_ref[r], preferred_element_type=jnp.float32)


def _inv_body(d0_ref, d1_ref, o_ref):
    o_ref[...] = 1.0 / (d0_ref[...] + d1_ref[...] + 1e-16)


def _post_body(a0_ref, a1_ref, x_ref, m_ref, wa_ref, ba_ref, al_ref, o_ref):
    g = a0_ref[...] + a1_ref[...]
    g = 0.5 * g * (1.0 + lax.erf(g * np.float32(1.0 / np.sqrt(2.0))))
    ba = ba_ref[...]
    al = al_ref[...]
    acc = jnp.zeros_like(g)
    alph = jnp.zeros_like(g)
    for t in range(T):
        y = jnp.dot(g, wa_ref[t], preferred_element_type=jnp.float32)
        mt = m_ref[:, t:t + 1]
        acc = acc + mt * (y + ba[t][None, :])
        alph = alph + mt * al[t][None, :]
    o_ref[...] = acc * alph + x_ref[...] * (1.0 - alph)


def _row_spec(shape):
    if shape[0] == N:
        return pl.BlockSpec((BN,) + shape[1:],
                            lambda i: (i,) + (0,) * (len(shape) - 1))
    return pl.BlockSpec((shape[0], BN) + shape[2:], lambda i: (0, i, 0))


def _full_spec(shape):
    return pl.BlockSpec(shape, lambda i: (0,) * len(shape))


def _tc_call(body, in_arrays, out_shapes, row_in, row_out):
    in_specs = [(_row_spec(a.shape) if rb else _full_spec(a.shape))
                for a, rb in zip(in_arrays, row_in)]
    out_specs = [(_row_spec(s.shape) if rb else _full_spec(s.shape))
                 for s, rb in zip(out_shapes, row_out)]
    return pl.pallas_call(
        body,
        grid=(GRID,),
        in_specs=in_specs,
        out_specs=out_specs if len(out_specs) > 1 else out_specs[0],
        out_shape=out_shapes if len(out_shapes) > 1 else out_shapes[0],
    )(*in_arrays)


def _worker_id():
    return lax.axis_index("s") * NC + lax.axis_index("c")


def _nchunks(wid):
    return BASE_CH + jnp.where(wid < EXTRA_W, 1, 0)


_IOTA16 = lambda: lax.iota(jnp.int32, 16)


def _sc_index_body(src_hbm, dst_hbm, rt_hbm, tt_hbm, nt16_hbm,
                   ikt_hbm, iq_hbm, ivm_hbm, ivt_hbm,
                   srcb, dstb, rtb, ttb, ntb, iktb, iqb, ivmb, ivtb, sem):
    wid = _worker_id()

    def chunk(i, carry):
        base = (wid + NW * i) * CE
        pltpu.sync_copy(src_hbm.at[pl.ds(base, CE)], srcb)
        pltpu.sync_copy(dst_hbm.at[pl.ds(base, CE)], dstb)
        pltpu.sync_copy(rt_hbm.at[pl.ds(base, CE)], rtb)
        pltpu.sync_copy(tt_hbm.at[pl.ds(base, CE)], ttb)
        pltpu.async_copy(nt16_hbm.at[srcb], ntb, sem).wait()
        for g in range(CE // 16):
            sl = pl.ds(g * 16, 16)
            rows = jnp.full((16,), g * 16, jnp.int32) + _IOTA16()
            ntv = plsc.load_gather(ntb, [rows, jnp.zeros((16,), jnp.int32)])
            ikt = ntv * MAXT + ttb[sl]
            iktb[sl] = ikt
            iqb[sl] = rtb[sl] * N + dstb[sl]
            ivmb[sl] = rtb[sl] * N + srcb[sl]
            ivtb[sl] = rtb[sl] * (T * MAXT) + ikt
        pltpu.sync_copy(iktb, ikt_hbm.at[pl.ds(base, CE)])
        pltpu.sync_copy(iqb, iq_hbm.at[pl.ds(base, CE)])
        pltpu.sync_copy(ivmb, ivm_hbm.at[pl.ds(base, CE)])
        pltpu.sync_copy(ivtb, ivt_hbm.at[pl.ds(base, CE)])
        return carry

    lax.fori_loop(0, _nchunks(wid), chunk, 0)


NQUAD1 = (BASE_CH + 1 + 3) // 4
NQUAD2 = (BASE_CH2 + 1 + 3) // 4


def _sc_att_body(kn_hbm, kt_hbm, qr_hbm, src_hbm, ikt_hbm, iq_hbm, dst_hbm,
                 zero_hbm,
                 ex_hbm, den0_hbm, den1_hbm,
                 srcb0, iktb0, iqb0, dstb0,
                 srcb1, iktb1, iqb1, dstb1,
                 srcb2, iktb2, iqb2, dstb2,
                 srcb3, iktb3, iqb3, dstb3,
                 knb0, ktb0, qrb0, knb1, ktb1, qrb1, exb0, exb1,
                 den_sh,
                 gsem0, gsem1, stsem0, stsem1):
    c = lax.axis_index("c")
    s = lax.axis_index("s")
    wid = s * NC + c
    nch = _nchunks(wid)
    idxs = [(srcb0, iktb0, iqb0, dstb0), (srcb1, iktb1, iqb1, dstb1),
            (srcb2, iktb2, iqb2, dstb2), (srcb3, iktb3, iqb3, dstb3)]
    gbufs = [(knb0, ktb0, qrb0), (knb1, ktb1, qrb1)]
    exs = [exb0, exb1]
    gsems = [gsem0, gsem1]
    stsems = [stsem0, stsem1]
    ihbm = (src_hbm, ikt_hbm, iq_hbm, dst_hbm)

    def load_idx(i, q):
        b = (wid + NW * i) * CE
        for href, vref in zip(ihbm, idxs[q]):
            pltpu.sync_copy(href.at[pl.ds(b, CE)], vref)

    def gathers(q, j):
        srcb, iktb, iqb, _ = idxs[q]
        knb, ktb, qrb = gbufs[j]
        return [pltpu.async_copy(kn_hbm.at[srcb], knb, gsems[j]),
                pltpu.async_copy(kt_hbm.at[iktb], ktb, gsems[j]),
                pltpu.async_copy(qr_hbm.at[iqb], qrb, gsems[j])]

    def compute(j):
        knb, ktb, qrb = gbufs[j]
        exb = exs[j]

        def edge(e2, carry):
            iota = _IOTA16()
            for u in range(2):
                ev = jnp.full((16,), e2 * 2 + u, jnp.int32)
                ex = jnp.zeros((16,), jnp.float32)
                for h in range(H):
                    col = iota + h * DK
                    kv = plsc.load_gather(knb, [ev, col])
                    tv = plsc.load_gather(ktb, [ev, col])
                    qv = plsc.load_gather(qrb, [ev, col])
                    sdot = jnp.sum((kv + tv) * qv)
                    ex = jnp.where(iota == h, sdot, ex)
                plsc.store_scatter(exb, [ev, iota], jnp.exp(ex))
            return carry

        lax.fori_loop(0, CE // 2, edge, 0)

    def store(b, q, j):
        d = pltpu.async_copy(exs[j], ex_hbm.at[pl.ds(b, CE)], stsems[j])
        pltpu.sync_copy(exs[j], den_sh.at[idxs[q][3]], add=True)
        return d

    rsl = pl.ds(s * RPS, RPS)
    pltpu.sync_copy(zero_hbm.at[rsl], den_sh.at[rsl])
    plsc.subcore_barrier()

    def quad(t, carry):
        cc = [4 * t + k for k in range(4)]
        bases = [(wid + NW * c_) * CE for c_ in cc]
        dg = [None] * 4
        ds = [None] * 4

        def when(k):
            return pl.when(cc[k] < nch)

        @when(0)
        def _():
            load_idx(cc[0], 0)
            dg[0] = gathers(0, 0)

        @when(1)
        def _():
            load_idx(cc[1], 1)
            dg[1] = gathers(1, 1)

        @when(0)
        def _():
            for d in dg[0]:
                d.wait()
            compute(0)
            ds[0] = store(bases[0], 0, 0)

        @when(2)
        def _():
            load_idx(cc[2], 2)
            dg[2] = gathers(2, 0)

        @when(1)
        def _():
            for d in dg[1]:
                d.wait()
            compute(1)
            ds[1] = store(bases[1], 1, 1)

        @when(3)
        def _():
            load_idx(cc[3], 3)
            dg[3] = gathers(3, 1)

        @when(0)
        def _():
            ds[0].wait()

        @when(2)
        def _():
            for d in dg[2]:
                d.wait()
            compute(0)
            ds[2] = store(bases[2], 2, 0)

        @when(1)
        def _():
            ds[1].wait()

        @when(3)
        def _():
            for d in dg[3]:
                d.wait()
            compute(1)
            ds[3] = store(bases[3], 3, 1)

        @when(2)
        def _():
            ds[2].wait()

        @when(3)
        def _():
            ds[3].wait()

        return carry

    lax.fori_loop(0, NQUAD1, quad, 0)
    plsc.subcore_barrier()

    @pl.when(c == 0)
    def _():
        pltpu.sync_copy(den_sh.at[rsl], den0_hbm.at[rsl])

    @pl.when(c == 1)
    def _():
        pltpu.sync_copy(den_sh.at[rsl], den1_hbm.at[rsl])


def _sc_agg_body(vmr_hbm, vmt_hbm, ex_hbm, inv_hbm,
                 ivm_hbm, ivt_hbm, dst_hbm, zero_hbm,
                 agg0_hbm, agg1_hbm,
                 ivmb0, ivtb0, dstb0, exb0,
                 ivmb1, ivtb1, dstb1, exb1,
                 ivmb2, ivtb2, dstb2, exb2,
                 ivmb3, ivtb3, dstb3, exb3,
                 vmrb0, vmtb0, invb0, vmrb1, vmtb1, invb1,
                 agg_sh,
                 gsem0, gsem1, stsem0, stsem1):
    c = lax.axis_index("c")
    s = lax.axis_index("s")
    wid = s * NC + c
    nch = BASE_CH2 + jnp.where(wid < EXTRA_W2, 1, 0)
    idxs = [(ivmb0, ivtb0, dstb0, exb0), (ivmb1, ivtb1, dstb1, exb1),
            (ivmb2, ivtb2, dstb2, exb2), (ivmb3, ivtb3, dstb3, exb3)]
    gbufs = [(vmrb0, vmtb0, invb0), (vmrb1, vmtb1, invb1)]
    gsems = [gsem0, gsem1]
    ihbm = (ivm_hbm, ivt_hbm, dst_hbm)

    def load_idx(i, q):
        b = (wid + NW * i) * CE2
        for href, vref in zip(ihbm, idxs[q][:3]):
            pltpu.sync_copy(href.at[pl.ds(b, CE2)], vref)
        pltpu.sync_copy(ex_hbm.at[pl.ds(b, CE2)], idxs[q][3])

    def gathers(q, j):
        ivmb, ivtb, dstb, _ = idxs[q]
        vmrb, vmtb, invb = gbufs[j]
        return [pltpu.async_copy(vmr_hbm.at[ivmb], vmrb, gsems[j]),
                pltpu.async_copy(vmt_hbm.at[ivtb], vmtb, gsems[j]),
                pltpu.async_copy(inv_hbm.at[dstb], invb, gsems[j])]

    def compute(q, j):
        vmrb, vmtb, invb = gbufs[j]
        exb = idxs[q][3]

        def edge(e2, carry):
            iota = _IOTA16()
            for u in range(2):
                ev = jnp.full((16,), e2 * 2 + u, jnp.int32)
                exv = plsc.load_gather(exb, [ev, iota])
                attn = exv * plsc.load_gather(invb, [ev, iota])
                for h in range(H):
                    col = iota + h * DK
                    mv = (plsc.load_gather(vmrb, [ev, col])
                          + plsc.load_gather(vmtb, [ev, col]))
                    plsc.store_scatter(vmrb, [ev, col], mv * attn[h])
            return carry

        lax.fori_loop(0, CE2 // 2, edge, 0)

    stsems = [stsem0, stsem1]

    def store(q, j):
        return pltpu.async_copy(gbufs[j][0], agg_sh.at[idxs[q][2]],
                                stsems[j], add=True)

    rsl = pl.ds(s * RPS, RPS)
    pltpu.sync_copy(zero_hbm.at[rsl], agg_sh.at[rsl])
    plsc.subcore_barrier()

    def quad(t, carry):
        cc = [4 * t + k for k in range(4)]
        dg = [None] * 4
        ds = [None] * 4

        def when(k):
            return pl.when(cc[k] < nch)

        @when(0)
        def _():
            load_idx(cc[0], 0)
            dg[0] = gathers(0, 0)

        @when(1)
        def _():
            load_idx(cc[1], 1)
            dg[1] = gathers(1, 1)

        @when(0)
        def _():
            for d in dg[0]:
                d.wait()
            compute(0, 0)
            ds[0] = store(0, 0)

        @when(2)
        def _():
            load_idx(cc[2], 2)

        @when(0)
        def _():
            ds[0].wait()

        @when(2)
        def _():
            dg[2] = gathers(2, 0)

        @when(1)
        def _():
            for d in dg[1]:
                d.wait()
            compute(1, 1)
            ds[1] = store(1, 1)

        @when(3)
        def _():
            load_idx(cc[3], 3)

        @when(1)
        def _():
            ds[1].wait()

        @when(3)
        def _():
            dg[3] = gathers(3, 1)

        @when(2)
        def _():
            for d in dg[2]:
                d.wait()
            compute(2, 0)
            ds[2] = store(2, 0)

        @when(3)
        def _():
            for d in dg[3]:
                d.wait()
            compute(3, 1)
            ds[3] = store(3, 1)

        @when(2)
        def _():
            ds[2].wait()

        @when(3)
        def _():
            ds[3].wait()

        return carry

    lax.fori_loop(0, NQUAD2, quad, 0)
    plsc.subcore_barrier()

    @pl.when(c == 0)
    def _():
        pltpu.sync_copy(agg_sh.at[rsl], agg0_hbm.at[rsl])

    @pl.when(c == 1)
    def _():
        pltpu.sync_copy(agg_sh.at[rsl], agg1_hbm.at[rsl])


@functools.cache
def _sc_index():
    return pl.kernel(
        _sc_index_body,
        out_type=[jax.ShapeDtypeStruct((E,), jnp.int32)] * 4,
        mesh=_mesh(),
        compiler_params=pltpu.CompilerParams(needs_layout_passes=False, use_tc_tiling_on_sc=False),
        scratch_types=(
            [pltpu.VMEM((CE,), jnp.int32)] * 4
            + [pltpu.VMEM((CE, 16), jnp.int32)]
            + [pltpu.VMEM((CE,), jnp.int32)] * 4
            + [pltpu.SemaphoreType.DMA]
        ),
    )


@functools.cache
def _sc_att():
    return pl.kernel(
        _sc_att_body,
        out_type=[jax.ShapeDtypeStruct((E, 16), jnp.float32),
                  jax.ShapeDtypeStruct((N, 16), jnp.float32),
                  jax.ShapeDtypeStruct((N, 16), jnp.float32)],
        mesh=_mesh(),
        compiler_params=pltpu.CompilerParams(needs_layout_passes=False, use_tc_tiling_on_sc=False),
        scratch_types=(
            [pltpu.VMEM((CE,), jnp.int32)] * 16
            + [pltpu.VMEM((CE, D), jnp.float32)] * 6
            + [pltpu.VMEM((CE, 16), jnp.float32)] * 2
            + [pltpu.VMEM_SHARED((N, 16), jnp.float32)]
            + [pltpu.SemaphoreType.DMA] * 4
        ),
    )


@functools.cache
def _sc_agg():
    return pl.kernel(
        _sc_agg_body,
        out_type=[jax.ShapeDtypeStruct((N, D), jnp.float32),
                  jax.ShapeDtypeStruct((N, D), jnp.float32)],
        mesh=_mesh(),
        compiler_params=pltpu.CompilerParams(needs_layout_passes=False, use_tc_tiling_on_sc=False),
        scratch_types=(
            ([pltpu.VMEM((CE2,), jnp.int32)] * 3
             + [pltpu.VMEM((CE2, 16), jnp.float32)]) * 4
            + ([pltpu.VMEM((CE2, D), jnp.float32)] * 2
               + [pltpu.VMEM((CE2, 16), jnp.float32)]) * 2
            + [pltpu.VMEM_SHARED((N, D), jnp.float32)]
            + [pltpu.SemaphoreType.DMA] * 4
        ),
    )


def _blockdiag(blocks):
    out = jnp.zeros((R, D, D), jnp.float32)
    for h in range(H):
        out = out.at[:, h * DK:(h + 1) * DK, h * DK:(h + 1) * DK].set(blocks[:, h])
    return out


def kernel(node_feature, params, node_type, edge_time, edge_index, edge_type):
    node_type = node_type.astype(jnp.int32)
    src = edge_index[0].astype(jnp.int32)
    dst = edge_index[1].astype(jnp.int32)
    rt = edge_type.astype(jnp.int32)
    tt = edge_time.astype(jnp.int32)

    mask = jnp.pad(jax.nn.one_hot(node_type, T, dtype=jnp.float32),
                   ((0, 0), (0, D - T)))
    nt16 = jnp.tile(node_type[:, None], (1, 16))
    zeros_d = jnp.zeros((N, 16), jnp.float32)
    zeros_a = jnp.zeros((N, D), jnp.float32)

    ikt, iq, ivm, ivt = _sc_index()(src, dst, rt, tt, nt16)

    x = _tc_call(
        _adapt_body,
        [node_feature, mask,
         params['adapt_W'].transpose(0, 2, 1), params['adapt_b']],
        [jax.ShapeDtypeStruct((N, D), jnp.float32)],
        [True, True, False, False], [True])

    for lp in params['layers']:
        rte_proj = _RTE @ lp['rte_W'].T + lp['rte_b']
        ktime = jnp.einsum('md,tod->tmo', rte_proj, lp['Wk']).reshape(T * MAXT, D)
        vtime = jnp.einsum('md,tod->tmo', rte_proj, lp['Wv']).reshape(T * MAXT, D)
        scale = (lp['rel_pri'] / np.sqrt(DK))[:, :, None, None]
        abig = _blockdiag(lp['rel_att'].transpose(0, 1, 3, 2) * scale)
        mbig = _blockdiag(lp['rel_msg'])
        vmt = jnp.einsum('md,rdo->rmo', vtime, mbig).reshape(R * T * MAXT, D)
        alpha = jnp.tile(jax.nn.sigmoid(lp['skip'])[:, None], (1, D))

        kn, qr, vmr = _tc_call(
            _prep_body,
            [x, mask,
             lp['Wk'].transpose(0, 2, 1), lp['bk'],
             lp['Wq'].transpose(0, 2, 1), lp['bq'],
             lp['Wv'].transpose(0, 2, 1), lp['bv'],
             abig, mbig],
            [jax.ShapeDtypeStruct((N, D), jnp.float32),
             jax.ShapeDtypeStruct((R, N, D), jnp.float32),
             jax.ShapeDtypeStruct((R, N, D), jnp.float32)],
            [True, True] + [False] * 8, [True, True, True])

        ex, den0, den1 = _sc_att()(kn, ktime, qr.reshape(R * N, D),
                                   src, ikt, iq, dst, zeros_d)
        inv = _tc_call(_inv_body, [den0, den1],
                       [jax.ShapeDtypeStruct((N, 16), jnp.float32)],
                       [True, True], [True])
        agg0, agg1 = _sc_agg()(vmr.reshape(R * N, D), vmt, ex, inv,
                               ivm, ivt, dst, zeros_a)

        x = _tc_call(
            _post_body,
            [agg0, agg1, x, mask,
             lp['Wa'].transpose(0, 2, 1), lp['ba'], alpha],
            [jax.ShapeDtypeStruct((N, D), jnp.float32)],
            [True, True, True, True, False, False, False], [True])

    return x


# speedup vs baseline: 1.2032x; 1.1776x over previous
import functools

import jax
import jax.numpy as jnp
import numpy as np
from jax import lax
from jax.experimental import pallas as pl
from jax.experimental.pallas import tpu as pltpu
from jax.experimental.pallas import tpu_sc as plsc

N = 10000
E = 160000
D = 128
T = 3
R = 5
H = 8
DK = 16
MAXT = 240

NC = 2
NS = 16
NW = NC * NS
CE = 128
NCHUNK = E // CE
BASE_CH = NCHUNK // NW
EXTRA_W = NCHUNK - BASE_CH * NW
CE2 = 64
NCHUNK2 = E // CE2
BASE_CH2 = NCHUNK2 // NW
EXTRA_W2 = NCHUNK2 - BASE_CH2 * NW
RPS = N // NS

BN = 1000
GRID = N // BN

@functools.cache
def _mesh():
    return plsc.VectorSubcoreMesh(core_axis_name="c", subcore_axis_name="s",
                                  num_cores=NC, num_subcores=NS)


def _rte_table():
    pos = np.arange(MAXT, dtype=np.float32)[:, None]
    div = np.exp(np.arange(0, D, 2, dtype=np.float32) * -(np.log(10000.0) / D))
    tab = np.zeros((MAXT, D), dtype=np.float32)
    tab[:, 0::2] = np.sin(pos * div)
    tab[:, 1::2] = np.cos(pos * div)
    return tab / np.sqrt(D)


_RTE = _rte_table()


def _typed_lin(xb, m_ref, w_ref, b_all):
    acc = jnp.zeros_like(xb)
    for t in range(T):
        y = jnp.dot(xb, w_ref[t], preferred_element_type=jnp.float32)
        acc = acc + m_ref[:, t:t + 1] * (y + b_all[t][None, :])
    return acc


def _adapt_body(x_ref, m_ref, w_ref, b_ref, o_ref):
    o_ref[...] = jnp.tanh(_typed_lin(x_ref[...], m_ref, w_ref, b_ref[...]))


def _prep_body(x_ref, m_ref, wk_ref, bk_ref, wq_ref, bq_ref, wv_ref, bv_ref,
               a_ref, mm_ref, kn_ref, qr_ref, vmr_ref):
    xb = x_ref[...]
    kn_ref[...] = _typed_lin(xb, m_ref, wk_ref, bk_ref[...])
    qb = _typed_lin(xb, m_ref, wq_ref, bq_ref[...])
    vb = _typed_lin(xb, m_ref, wv_ref, bv_ref[...])
    for r in range(R):
        qr_ref[r] = jnp.dot(qb, a_ref[r], preferred_element_type=jnp.float32)
        vmr_ref[r] = jnp.dot(vb, mm_ref[r], preferred_element_type=jnp.float32)


def _inv_body(d0_ref, d1_ref, o_ref):
    o_ref[...] = 1.0 / (d0_ref[...] + d1_ref[...] + 1e-16)


def _post_body(a0_ref, a1_ref, x_ref, m_ref, wa_ref, ba_ref, al_ref, o_ref):
    g = a0_ref[...] + a1_ref[...]
    g = 0.5 * g * (1.0 + lax.erf(g * np.float32(1.0 / np.sqrt(2.0))))
    ba = ba_ref[...]
    al = al_ref[...]
    acc = jnp.zeros_like(g)
    alph = jnp.zeros_like(g)
    for t in range(T):
        y = jnp.dot(g, wa_ref[t], preferred_element_type=jnp.float32)
        mt = m_ref[:, t:t + 1]
        acc = acc + mt * (y + ba[t][None, :])
        alph = alph + mt * al[t][None, :]
    o_ref[...] = acc * alph + x_ref[...] * (1.0 - alph)


def _row_spec(shape):
    if shape[0] == N:
        return pl.BlockSpec((BN,) + shape[1:],
                            lambda i: (i,) + (0,) * (len(shape) - 1))
    return pl.BlockSpec((shape[0], BN) + shape[2:], lambda i: (0, i, 0))


def _full_spec(shape):
    return pl.BlockSpec(shape, lambda i: (0,) * len(shape))


def _tc_call(body, in_arrays, out_shapes, row_in, row_out):
    in_specs = [(_row_spec(a.shape) if rb else _full_spec(a.shape))
                for a, rb in zip(in_arrays, row_in)]
    out_specs = [(_row_spec(s.shape) if rb else _full_spec(s.shape))
                 for s, rb in zip(out_shapes, row_out)]
    return pl.pallas_call(
        body,
        grid=(GRID,),
        in_specs=in_specs,
        out_specs=out_specs if len(out_specs) > 1 else out_specs[0],
        out_shape=out_shapes if len(out_shapes) > 1 else out_shapes[0],
    )(*in_arrays)


def _worker_id():
    return lax.axis_index("s") * NC + lax.axis_index("c")


def _nchunks(wid):
    return BASE_CH + jnp.where(wid < EXTRA_W, 1, 0)


_IOTA16 = lambda: lax.iota(jnp.int32, 16)


def _sc_index_body(src_hbm, dst_hbm, rt_hbm, tt_hbm, nt16_hbm,
                   ikt_hbm, iq_hbm, ivm_hbm, ivt_hbm,
                   srcb, dstb, rtb, ttb, ntb, iktb, iqb, ivmb, ivtb, sem):
    wid = _worker_id()

    def chunk(i, carry):
        base = (wid + NW * i) * CE
        dl = [pltpu.async_copy(h.at[pl.ds(base, CE)], v, sem)
              for h, v in ((src_hbm, srcb), (dst_hbm, dstb),
                           (rt_hbm, rtb), (tt_hbm, ttb))]
        for d in dl:
            d.wait()
        pltpu.async_copy(nt16_hbm.at[srcb], ntb, sem).wait()
        for g in range(CE // 16):
            sl = pl.ds(g * 16, 16)
            rows = jnp.full((16,), g * 16, jnp.int32) + _IOTA16()
            ntv = plsc.load_gather(ntb, [rows, jnp.zeros((16,), jnp.int32)])
            ikt = ntv * MAXT + ttb[sl]
            iktb[sl] = ikt
            iqb[sl] = rtb[sl] * N + dstb[sl]
            ivmb[sl] = rtb[sl] * N + srcb[sl]
            ivtb[sl] = rtb[sl] * (T * MAXT) + ikt
        dw = [pltpu.async_copy(v, h.at[pl.ds(base, CE)], sem)
              for v, h in ((iktb, ikt_hbm), (iqb, iq_hbm),
                           (ivmb, ivm_hbm), (ivtb, ivt_hbm))]
        for d in dw:
            d.wait()
        return carry

    lax.fori_loop(0, _nchunks(wid), chunk, 0)


NQUAD1 = (BASE_CH + 1 + 3) // 4
NQUAD2 = (BASE_CH2 + 1 + 3) // 4


def _sc_att_body(kn_hbm, kt_hbm, qr_hbm, src_hbm, ikt_hbm, iq_hbm, dst_hbm,
                 zero_hbm,
                 ex_hbm, den0_hbm, den1_hbm,
                 srcb0, iktb0, iqb0, dstb0,
                 srcb1, iktb1, iqb1, dstb1,
                 srcb2, iktb2, iqb2, dstb2,
                 srcb3, iktb3, iqb3, dstb3,
                 knb0, ktb0, qrb0, knb1, ktb1, qrb1, exb0, exb1,
                 den_sh,
                 gsem0, gsem1, stsem0, stsem1, ixsem):
    c = lax.axis_index("c")
    s = lax.axis_index("s")
    wid = s * NC + c
    nch = _nchunks(wid)
    idxs = [(srcb0, iktb0, iqb0, dstb0), (srcb1, iktb1, iqb1, dstb1),
            (srcb2, iktb2, iqb2, dstb2), (srcb3, iktb3, iqb3, dstb3)]
    gbufs = [(knb0, ktb0, qrb0), (knb1, ktb1, qrb1)]
    exs = [exb0, exb1]
    gsems = [gsem0, gsem1]
    stsems = [stsem0, stsem1]
    ihbm = (src_hbm, ikt_hbm, iq_hbm, dst_hbm)

    def load_idx(i, q):
        b = (wid + NW * i) * CE
        dl = [pltpu.async_copy(href.at[pl.ds(b, CE)], vref, ixsem)
              for href, vref in zip(ihbm, idxs[q])]
        for d in dl:
            d.wait()

    def gathers(q, j):
        srcb, iktb, iqb, _ = idxs[q]
        knb, ktb, qrb = gbufs[j]
        return [pltpu.async_copy(kn_hbm.at[srcb], knb, gsems[j]),
                pltpu.async_copy(kt_hbm.at[iktb], ktb, gsems[j]),
                pltpu.async_copy(qr_hbm.at[iqb], qrb, gsems[j])]

    def compute(j):
        knb, ktb, qrb = gbufs[j]
        exb = exs[j]

        def edge(e2, carry):
            iota = _IOTA16()
            for u in range(2):
                ev = jnp.full((16,), e2 * 2 + u, jnp.int32)
                ex = jnp.zeros((16,), jnp.float32)
                for h in range(H):
                    col = iota + h * DK
                    kv = plsc.load_gather(knb, [ev, col])
                    tv = plsc.load_gather(ktb, [ev, col])
                    qv = plsc.load_gather(qrb, [ev, col])
                    sdot = jnp.sum((kv + tv) * qv)
                    ex = jnp.where(iota == h, sdot, ex)
                plsc.store_scatter(exb, [ev, iota], jnp.exp(ex))
            return carry

        lax.fori_loop(0, CE // 2, edge, 0)

    def store(b, q, j):
        d = pltpu.async_copy(exs[j], ex_hbm.at[pl.ds(b, CE)], stsems[j])
        pltpu.sync_copy(exs[j], den_sh.at[idxs[q][3]], add=True)
        return d

    rsl = pl.ds(s * RPS, RPS)
    pltpu.sync_copy(zero_hbm.at[rsl], den_sh.at[rsl])
    plsc.subcore_barrier()

    def quad(t, carry):
        cc = [4 * t + k for k in range(4)]
        bases = [(wid + NW * c_) * CE for c_ in cc]
        dg = [None] * 4
        ds = [None] * 4

        def when(k):
            return pl.when(cc[k] < nch)

        @when(0)
        def _():
            load_idx(cc[0], 0)
            dg[0] = gathers(0, 0)

        @when(1)
        def _():
            load_idx(cc[1], 1)
            dg[1] = gathers(1, 1)

        @when(0)
        def _():
            for d in dg[0]:
                d.wait()
            compute(0)
            ds[0] = store(bases[0], 0, 0)

        @when(2)
        def _():
            load_idx(cc[2], 2)
            dg[2] = gathers(2, 0)

        @when(1)
        def _():
            for d in dg[1]:
                d.wait()
            compute(1)
            ds[1] = store(bases[1], 1, 1)

        @when(3)
        def _():
            load_idx(cc[3], 3)
            dg[3] = gathers(3, 1)

        @when(0)
        def _():
            ds[0].wait()

        @when(2)
        def _():
            for d in dg[2]:
                d.wait()
            compute(0)
            ds[2] = store(bases[2], 2, 0)

        @when(1)
        def _():
            ds[1].wait()

        @when(3)
        def _():
            for d in dg[3]:
                d.wait()
            compute(1)
            ds[3] = store(bases[3], 3, 1)

        @when(2)
        def _():
            ds[2].wait()

        @when(3)
        def _():
            ds[3].wait()

        return carry

    lax.fori_loop(0, NQUAD1, quad, 0)
    plsc.subcore_barrier()

    @pl.when(c == 0)
    def _():
        pltpu.sync_copy(den_sh.at[rsl], den0_hbm.at[rsl])

    @pl.when(c == 1)
    def _():
        pltpu.sync_copy(den_sh.at[rsl], den1_hbm.at[rsl])


def _sc_agg_body(vmr_hbm, vmt_hbm, ex_hbm, inv_hbm,
                 ivm_hbm, ivt_hbm, dst_hbm, zero_hbm,
                 agg0_hbm, agg1_hbm,
                 ivmb0, ivtb0, dstb0, exb0,
                 ivmb1, ivtb1, dstb1, exb1,
                 ivmb2, ivtb2, dstb2, exb2,
                 ivmb3, ivtb3, dstb3, exb3,
                 vmrb0, vmtb0, invb0, vmrb1, vmtb1, invb1,
                 agg_sh,
                 gsem0, gsem1, stsem0, stsem1, ixsem):
    c = lax.axis_index("c")
    s = lax.axis_index("s")
    wid = s * NC + c
    nch = BASE_CH2 + jnp.where(wid < EXTRA_W2, 1, 0)
    idxs = [(ivmb0, ivtb0, dstb0, exb0), (ivmb1, ivtb1, dstb1, exb1),
            (ivmb2, ivtb2, dstb2, exb2), (ivmb3, ivtb3, dstb3, exb3)]
    gbufs = [(vmrb0, vmtb0, invb0), (vmrb1, vmtb1, invb1)]
    gsems = [gsem0, gsem1]
    ihbm = (ivm_hbm, ivt_hbm, dst_hbm)

    def load_idx(i, q):
        b = (wid + NW * i) * CE2
        dl = [pltpu.async_copy(href.at[pl.ds(b, CE2)], vref, ixsem)
              for href, vref in zip(ihbm, idxs[q][:3])]
        dl.append(pltpu.async_copy(ex_hbm.at[pl.ds(b, CE2)], idxs[q][3],
                                   ixsem))
        for d in dl:
            d.wait()

    def gathers(q, j):
        ivmb, ivtb, dstb, _ = idxs[q]
        vmrb, vmtb, invb = gbufs[j]
        return [pltpu.async_copy(vmr_hbm.at[ivmb], vmrb, gsems[j]),
                pltpu.async_copy(vmt_hbm.at[ivtb], vmtb, gsems[j]),
                pltpu.async_copy(inv_hbm.at[dstb], invb, gsems[j])]

    def compute(q, j):
        vmrb, vmtb, invb = gbufs[j]
        exb = idxs[q][3]

        def edge(e2, carry):
            iota = _IOTA16()
            for u in range(2):
                ev = jnp.full((16,), e2 * 2 + u, jnp.int32)
                exv = plsc.load_gather(exb, [ev, iota])
                attn = exv * plsc.load_gather(invb, [ev, iota])
                for h in range(H):
                    col = iota + h * DK
                    mv = (plsc.load_gather(vmrb, [ev, col])
                          + plsc.load_gather(vmtb, [ev, col]))
                    plsc.store_scatter(vmrb, [ev, col], mv * attn[h])
            return carry

        lax.fori_loop(0, CE2 // 2, edge, 0)

    stsems = [stsem0, stsem1]

    def store(q, j):
        return pltpu.async_copy(gbufs[j][0], agg_sh.at[idxs[q][2]],
                                stsems[j], add=True)

    rsl = pl.ds(s * RPS, RPS)
    pltpu.sync_copy(zero_hbm.at[rsl], agg_sh.at[rsl])
    plsc.subcore_barrier()

    def quad(t, carry):
        cc = [4 * t + k for k in range(4)]
        dg = [None] * 4
        ds = [None] * 4

        def when(k):
            return pl.when(cc[k] < nch)

        @when(0)
        def _():
            load_idx(cc[0], 0)
            dg[0] = gathers(0, 0)

        @when(1)
        def _():
            load_idx(cc[1], 1)
            dg[1] = gathers(1, 1)

        @when(0)
        def _():
            for d in dg[0]:
                d.wait()
            compute(0, 0)
            ds[0] = store(0, 0)

        @when(2)
        def _():
            load_idx(cc[2], 2)

        @when(0)
        def _():
            ds[0].wait()

        @when(2)
        def _():
            dg[2] = gathers(2, 0)

        @when(1)
        def _():
            for d in dg[1]:
                d.wait()
            compute(1, 1)
            ds[1] = store(1, 1)

        @when(3)
        def _():
            load_idx(cc[3], 3)

        @when(1)
        def _():
            ds[1].wait()

        @when(3)
        def _():
            dg[3] = gathers(3, 1)

        @when(2)
        def _():
            for d in dg[2]:
                d.wait()
            compute(2, 0)
            ds[2] = store(2, 0)

        @when(3)
        def _():
            for d in dg[3]:
                d.wait()
            compute(3, 1)
            ds[3] = store(3, 1)

        @when(2)
        def _():
            ds[2].wait()

        @when(3)
        def _():
            ds[3].wait()

        return carry

    lax.fori_loop(0, NQUAD2, quad, 0)
    plsc.subcore_barrier()

    @pl.when(c == 0)
    def _():
        pltpu.sync_copy(agg_sh.at[rsl], agg0_hbm.at[rsl])

    @pl.when(c == 1)
    def _():
        pltpu.sync_copy(agg_sh.at[rsl], agg1_hbm.at[rsl])


@functools.cache
def _sc_index():
    return pl.kernel(
        _sc_index_body,
        out_type=[jax.ShapeDtypeStruct((E,), jnp.int32)] * 4,
        mesh=_mesh(),
        compiler_params=pltpu.CompilerParams(needs_layout_passes=False, use_tc_tiling_on_sc=False),
        scratch_types=(
            [pltpu.VMEM((CE,), jnp.int32)] * 4
            + [pltpu.VMEM((CE, 16), jnp.int32)]
            + [pltpu.VMEM((CE,), jnp.int32)] * 4
            + [pltpu.SemaphoreType.DMA]
        ),
    )


@functools.cache
def _sc_att():
    return pl.kernel(
        _sc_att_body,
        out_type=[jax.ShapeDtypeStruct((E, 16), jnp.float32),
                  jax.ShapeDtypeStruct((N, 16), jnp.float32),
                  jax.ShapeDtypeStruct((N, 16), jnp.float32)],
        mesh=_mesh(),
        compiler_params=pltpu.CompilerParams(needs_layout_passes=False, use_tc_tiling_on_sc=False),
        scratch_types=(
            [pltpu.VMEM((CE,), jnp.int32)] * 16
            + [pltpu.VMEM((CE, D), jnp.float32)] * 6
            + [pltpu.VMEM((CE, 16), jnp.float32)] * 2
            + [pltpu.VMEM_SHARED((N, 16), jnp.float32)]
            + [pltpu.SemaphoreType.DMA] * 5
        ),
    )


@functools.cache
def _sc_agg():
    return pl.kernel(
        _sc_agg_body,
        out_type=[jax.ShapeDtypeStruct((N, D), jnp.float32),
                  jax.ShapeDtypeStruct((N, D), jnp.float32)],
        mesh=_mesh(),
        compiler_params=pltpu.CompilerParams(needs_layout_passes=False, use_tc_tiling_on_sc=False),
        scratch_types=(
            ([pltpu.VMEM((CE2,), jnp.int32)] * 3
             + [pltpu.VMEM((CE2, 16), jnp.float32)]) * 4
            + ([pltpu.VMEM((CE2, D), jnp.float32)] * 2
               + [pltpu.VMEM((CE2, 16), jnp.float32)]) * 2
            + [pltpu.VMEM_SHARED((N, D), jnp.float32)]
            + [pltpu.SemaphoreType.DMA] * 5
        ),
    )


def _blockdiag(blocks):
    out = jnp.zeros((R, D, D), jnp.float32)
    for h in range(H):
        out = out.at[:, h * DK:(h + 1) * DK, h * DK:(h + 1) * DK].set(blocks[:, h])
    return out


def kernel(node_feature, params, node_type, edge_time, edge_index, edge_type):
    node_type = node_type.astype(jnp.int32)
    src = edge_index[0].astype(jnp.int32)
    dst = edge_index[1].astype(jnp.int32)
    rt = edge_type.astype(jnp.int32)
    tt = edge_time.astype(jnp.int32)

    mask = jnp.pad(jax.nn.one_hot(node_type, T, dtype=jnp.float32),
                   ((0, 0), (0, D - T)))
    nt16 = jnp.tile(node_type[:, None], (1, 16))
    zeros_d = jnp.zeros((N, 16), jnp.float32)
    zeros_a = jnp.zeros((N, D), jnp.float32)

    ikt, iq, ivm, ivt = _sc_index()(src, dst, rt, tt, nt16)

    x = _tc_call(
        _adapt_body,
        [node_feature, mask,
         params['adapt_W'].transpose(0, 2, 1), params['adapt_b']],
        [jax.ShapeDtypeStruct((N, D), jnp.float32)],
        [True, True, False, False], [True])

    for lp in params['layers']:
        rte_proj = _RTE @ lp['rte_W'].T + lp['rte_b']
        ktime = jnp.einsum('md,tod->tmo', rte_proj, lp['Wk']).reshape(T * MAXT, D)
        vtime = jnp.einsum('md,tod->tmo', rte_proj, lp['Wv']).reshape(T * MAXT, D)
        scale = (lp['rel_pri'] / np.sqrt(DK))[:, :, None, None]
        abig = _blockdiag(lp['rel_att'].transpose(0, 1, 3, 2) * scale)
        mbig = _blockdiag(lp['rel_msg'])
        vmt = jnp.einsum('md,rdo->rmo', vtime, mbig).reshape(R * T * MAXT, D)
        alpha = jnp.tile(jax.nn.sigmoid(lp['skip'])[:, None], (1, D))

        kn, qr, vmr = _tc_call(
            _prep_body,
            [x, mask,
             lp['Wk'].transpose(0, 2, 1), lp['bk'],
             lp['Wq'].transpose(0, 2, 1), lp['bq'],
             lp['Wv'].transpose(0, 2, 1), lp['bv'],
             abig, mbig],
            [jax.ShapeDtypeStruct((N, D), jnp.float32),
             jax.ShapeDtypeStruct((R, N, D), jnp.float32),
             jax.ShapeDtypeStruct((R, N, D), jnp.float32)],
            [True, True] + [False] * 8, [True, True, True])

        ex, den0, den1 = _sc_att()(kn, ktime, qr.reshape(R * N, D),
                                   src, ikt, iq, dst, zeros_d)
        inv = _tc_call(_inv_body, [den0, den1],
                       [jax.ShapeDtypeStruct((N, 16), jnp.float32)],
                       [True, True], [True])
        agg0, agg1 = _sc_agg()(vmr.reshape(R * N, D), vmt, ex, inv,
                               ivm, ivt, dst, zeros_a)

        x = _tc_call(
            _post_body,
            [agg0, agg1, x, mask,
             lp['Wa'].transpose(0, 2, 1), lp['ba'], alpha],
            [jax.ShapeDtypeStruct((N, D), jnp.float32)],
            [True, True, True, True, False, False, False], [True])

    return x
